# baseline (speedup 1.0000x reference)
import functools

import jax
import jax.numpy as jnp
from jax import lax
from jax.experimental import pallas as pl
from jax.experimental.pallas import tpu as pltpu
from jax.experimental.pallas import tpu_sc as plsc

_NC = 2
_NS = 16
_L = 16
_NW = _NC * _NS


def _make_sc_kernel(B, D):
    bpw = B // _NW
    mesh = plsc.VectorSubcoreMesh(core_axis_name="c", subcore_axis_name="s")

    @functools.partial(
        pl.kernel,
        mesh=mesh,
        compiler_params=pltpu.CompilerParams(
            needs_layout_passes=False, use_tc_tiling_on_sc=False),
        out_type=jax.ShapeDtypeStruct((B,), jnp.float32),
        scratch_types=[
            pltpu.VMEM((bpw,), jnp.int32),
            pltpu.VMEM((bpw,), jnp.int32),
            pltpu.VMEM((bpw,), jnp.int32),
            pltpu.VMEM((bpw,), jnp.int32),
            pltpu.VMEM((bpw, D), jnp.float32),
            pltpu.VMEM((bpw, D), jnp.float32),
            pltpu.VMEM((bpw, _L), jnp.float32),
            pltpu.VMEM((bpw, _L), jnp.float32),
            pltpu.VMEM((bpw,), jnp.float32),
            pltpu.SemaphoreType.DMA,
        ],
    )
    def k(uid_hbm, hid_hbm, uemb_hbm, hemb_hbm, ub_hbm, hb_hbm, out_hbm,
          idx_u, idx_h, idx_u4, idx_h4, urows, hrows, ub_v, hb_v, out_v, sem):
        wid = lax.axis_index("s") * _NC + lax.axis_index("c")
        base = wid * bpw
        pltpu.sync_copy(uid_hbm.at[pl.ds(base, bpw)], idx_u)
        pltpu.sync_copy(hid_hbm.at[pl.ds(base, bpw)], idx_h)
        cu = pltpu.async_copy(uemb_hbm.at[idx_u], urows, sem)
        ch = pltpu.async_copy(hemb_hbm.at[idx_h], hrows, sem)

        def shift_body(g, carry):
            sl = pl.ds(g * _L, _L)
            idx_u4[sl] = lax.shift_right_logical(idx_u[sl], 4)
            idx_h4[sl] = lax.shift_right_logical(idx_h[sl], 4)
            return carry

        lax.fori_loop(0, bpw // _L, shift_body, 0)
        cub = pltpu.async_copy(ub_hbm.at[idx_u4], ub_v, sem)
        chb = pltpu.async_copy(hb_hbm.at[idx_h4], hb_v, sem)
        cu.wait()
        ch.wait()
        cub.wait()
        chb.wait()

        lane = lax.iota(jnp.int32, _L)
        fifteen = jnp.full((_L,), 15, jnp.int32)

        def body(g, carry):
            ids = g * _L + lane
            sl = pl.ds(g * _L, _L)
            acc = (plsc.load_gather(ub_v, [ids, idx_u[sl] & fifteen]) +
                   plsc.load_gather(hb_v, [ids, idx_h[sl] & fifteen]))
            for d in range(D):
                dcol = jnp.full((_L,), d, jnp.int32)
                acc = acc + plsc.load_gather(urows, [ids, dcol]) * plsc.load_gather(
                    hrows, [ids, dcol])
            out_v[sl] = acc
            return carry

        lax.fori_loop(0, bpw // _L, body, 0)
        pltpu.sync_copy(out_v, out_hbm.at[pl.ds(base, bpw)])

    return k


def kernel(user_id_input, hotel_id_input, user_embeddings, hotel_embeddings,
           user_biases, hotel_biases):
    B = user_id_input.shape[0]
    D = user_embeddings.shape[1]
    k = _make_sc_kernel(B, D)
    return k(user_id_input.astype(jnp.int32), hotel_id_input.astype(jnp.int32),
             user_embeddings, hotel_embeddings,
             user_biases.reshape(-1, _L), hotel_biases.reshape(-1, _L))
